# baseline (speedup 1.0000x reference)
import jax
import jax.numpy as jnp
import numpy as np
from jax.experimental import pallas as pl
from jax.experimental.pallas import tpu as pltpu

B = 4
S = 8192
D = 768
N1 = 64
N2 = 128
TOPK = 10
KPAD = 16
DB = 128
TB2 = 2048
TB3 = 2048


def _const_mats():
    t2 = np.arange(N2, dtype=np.float64)[:, None]
    lg = np.arange(N1, dtype=np.float64)[None, :]
    dft_h = np.concatenate(
        [np.cos(2.0 * np.pi * t2 * lg / N2),
         np.cos(np.pi * t2),
         -np.sin(2.0 * np.pi * t2 * lg[:, 1:] / N2)], axis=1).astype(np.float32)
    t1g = np.arange(N1, dtype=np.float64)[:, None, None]
    lg3 = np.arange(N1, dtype=np.float64)[None, None, :]
    angt = 2.0 * np.pi * t1g * lg3 / S
    c1 = np.cos(angt)
    s1 = np.sin(angt)
    c2 = c1.copy()
    c2[:, :, 0] = 0.0
    pmat = np.concatenate([c1, c2], axis=2).astype(np.float32)
    qmat = np.concatenate([s1, -s1], axis=2).astype(np.float32)
    f1 = np.arange(33, dtype=np.float64)
    t1 = np.arange(N1, dtype=np.float64)
    ang3 = 2.0 * np.pi * np.outer(f1, t1) / N1
    z7 = np.zeros((7, N1))
    cs = np.concatenate([np.cos(ang3), z7, np.sin(ang3), z7],
                        axis=0).astype(np.float32)
    angc = 2.0 * np.pi * np.outer(2.0 * f1 + 1.0, t1) / 256.0
    csc = np.concatenate([np.cos(angc), z7, np.sin(angc), z7],
                         axis=0).astype(np.float32)
    return dft_h, pmat, qmat, cs, csc


_DFTH, _PM, _QM, _CS, _CSC = _const_mats()


def _bsplit(a):
    hi = a.astype(jnp.bfloat16)
    lo = (a - hi.astype(jnp.float32)).astype(jnp.bfloat16)
    return hi, lo


def _dot3(a, b, dims):
    ah, al = _bsplit(a)
    bh, bl = _bsplit(b)

    def dd(u, v):
        return jax.lax.dot_general(u, v, (dims, ((), ())),
                                   preferred_element_type=jnp.float32)

    return dd(ah, bh) + dd(ah, bl) + dd(al, bh)


def _p1_kernel(x_ref, dft_ref, p_ref, q_ref, cs_ref, csc_ref,
               idx_ref, acc_ref):
    j = pl.program_id(1)
    nd = pl.num_programs(1)
    xb = x_ref[0]
    xb2 = xb.reshape(N2, N1 * DB)
    y = _dot3(xb2, dft_ref[...], ((0,), (0,)))
    y3 = y.reshape(N1, DB, N2)
    yp = y3 * p_ref[...] + jnp.roll(y3, N1, axis=2) * q_ref[...]
    mm = _dot3(cs_ref[...], yp, ((1,), (0,)))
    mmr = jnp.roll(mm, N1, axis=2)
    u = mm[:40] + mmr[40:]
    v = mmr[:40] - mm[40:]
    sab = jnp.sum(jnp.sqrt(u * u + v * v), axis=1)
    m3 = _dot3(csc_ref[...], y3[:, :, N1], ((1,), (0,)))
    sc = jnp.sum(jnp.sqrt(m3[:40] ** 2 + m3[40:] ** 2), axis=1)
    s = jnp.concatenate(
        [sab, sc[:, None], jnp.zeros((40, N2 - 1), jnp.float32)], axis=1)

    @pl.when(j == 0)
    def _():
        acc_ref[...] = s

    @pl.when(j > 0)
    def _():
        acc_ref[...] = acc_ref[...] + s

    @pl.when(j == nd - 1)
    def _():
        a = acc_ref[...]
        i0 = jax.lax.broadcasted_iota(jnp.int32, (40, 2 * N2), 0)
        i1 = jax.lax.broadcasted_iota(jnp.int32, (40, 2 * N2), 1)
        fa = N2 * i0 + i1
        fb = N2 * i0 - (i1 - N1)
        fc = N2 * i0 + N1
        fmat = jnp.where(i1 < N1, fa, jnp.where(i1 < 2 * N1, fb, fc))
        valid = ((((i1 < N1) & (fa <= S // 2))
                  | ((i1 >= N1 + 1) & (i1 < 2 * N1) & (i0 >= 1))
                  | ((i1 == 2 * N1) & (i0 <= 31)))
                 & (fmat <= S // 2))
        a = jnp.where(valid, a, -1.0)
        colid = jax.lax.broadcasted_iota(jnp.int32, (1, KPAD), 1)
        row = jnp.zeros((1, KPAD), jnp.int32)
        for k in range(TOPK):
            m = jnp.max(a)
            fk = jnp.min(jnp.where(a == m, fmat, jnp.int32(2 ** 20)))
            row = jnp.where(colid == k, fk, row)
            a = jnp.where(fmat == fk, -1.0, a)
        idx_ref[0] = row


def _basis_block(idx_row, t0, tb, ncols):
    f2x = jnp.concatenate([idx_row, idx_row], axis=1)
    tmat = t0 + jax.lax.broadcasted_iota(jnp.int32, (tb, ncols), 0)
    prod = tmat * f2x
    ang = (prod & (S - 1)).astype(jnp.float32) * (2.0 * np.pi / S)
    colid = jax.lax.broadcasted_iota(jnp.int32, (tb, ncols), 1)
    return jnp.where(colid < KPAD, jnp.cos(ang), jnp.sin(ang)), f2x, colid


def _p2_kernel(x_ref, idx_ref, w_ref, pq_ref, acc_ref):
    j = pl.program_id(1)
    nt = pl.num_programs(1)
    xb = x_ref[0]
    basis, _, _ = _basis_block(idx_ref[0], j * TB2, TB2, 2 * KPAD)
    ps = jax.lax.dot_general(basis.astype(jnp.bfloat16), xb.astype(jnp.bfloat16),
                             (((0,), (0,)), ((), ())),
                             preferred_element_type=jnp.float32)

    @pl.when(j == 0)
    def _():
        acc_ref[...] = ps

    @pl.when(j > 0)
    def _():
        acc_ref[...] = acc_ref[...] + ps

    @pl.when(j == nt - 1)
    def _():
        pq_ref[0] = _dot3(acc_ref[...], w_ref[...], ((1,), (1,)))


def _p3_kernel(pq_ref, idx_ref, bias_ref, y_ref):
    j = pl.program_id(1)
    basis, f2x, colid = _basis_block(idx_ref[0], j * TB3, TB3, 2 * KPAD)
    kid = colid & (KPAD - 1)
    cval = jnp.where((f2x == 0) | (f2x == S // 2), 1.0, 2.0)
    coef = jnp.where(kid < TOPK, cval, 0.0) * (1.0 / S)
    basis = basis * coef
    y = jax.lax.dot_general(basis.astype(jnp.bfloat16),
                            pq_ref[0].astype(jnp.bfloat16),
                            (((1,), (0,)), ((), ())),
                            preferred_element_type=jnp.float32)
    y_ref[0] = y + bias_ref[...]


@jax.jit
def kernel(x, W, b):
    x4 = x.reshape(B, N2, N1, D)
    nd = D // DB
    idx = pl.pallas_call(
        _p1_kernel,
        grid=(B, nd),
        in_specs=[
            pl.BlockSpec((1, N2, N1, DB), lambda bi, j: (bi, 0, 0, j)),
            pl.BlockSpec((N2, N2), lambda bi, j: (0, 0)),
            pl.BlockSpec((N1, 1, N2), lambda bi, j: (0, 0, 0)),
            pl.BlockSpec((N1, 1, N2), lambda bi, j: (0, 0, 0)),
            pl.BlockSpec((80, N1), lambda bi, j: (0, 0)),
            pl.BlockSpec((80, N1), lambda bi, j: (0, 0)),
        ],
        out_specs=pl.BlockSpec((1, 1, KPAD), lambda bi, j: (bi, 0, 0)),
        out_shape=jax.ShapeDtypeStruct((B, 1, KPAD), jnp.int32),
        scratch_shapes=[pltpu.VMEM((40, 2 * N2), jnp.float32)],
        compiler_params=pltpu.CompilerParams(
            dimension_semantics=("arbitrary", "arbitrary")),
    )(x4, _DFTH, _PM, _QM, _CS, _CSC)

    nt = S // TB2
    pq = pl.pallas_call(
        _p2_kernel,
        grid=(B, nt),
        in_specs=[
            pl.BlockSpec((1, TB2, D), lambda bi, j: (bi, j, 0)),
            pl.BlockSpec((1, 1, KPAD), lambda bi, j: (bi, 0, 0)),
            pl.BlockSpec((D, D), lambda bi, j: (0, 0)),
        ],
        out_specs=pl.BlockSpec((1, 2 * KPAD, D), lambda bi, j: (bi, 0, 0)),
        out_shape=jax.ShapeDtypeStruct((B, 2 * KPAD, D), jnp.float32),
        scratch_shapes=[pltpu.VMEM((2 * KPAD, D), jnp.float32)],
        compiler_params=pltpu.CompilerParams(
            dimension_semantics=("arbitrary", "arbitrary")),
    )(x, idx, W)

    nt3 = S // TB3
    y = pl.pallas_call(
        _p3_kernel,
        grid=(B, nt3),
        in_specs=[
            pl.BlockSpec((1, 2 * KPAD, D), lambda bi, j: (bi, 0, 0)),
            pl.BlockSpec((1, 1, KPAD), lambda bi, j: (bi, 0, 0)),
            pl.BlockSpec((1, D), lambda bi, j: (0, 0)),
        ],
        out_specs=pl.BlockSpec((1, TB3, D), lambda bi, j: (bi, j, 0)),
        out_shape=jax.ShapeDtypeStruct((B, S, D), jnp.float32),
        compiler_params=pltpu.CompilerParams(
            dimension_semantics=("arbitrary", "arbitrary")),
    )(pq, idx, b.reshape(1, D))
    return y


# speedup vs baseline: 1.0006x; 1.0006x over previous
import jax
import jax.numpy as jnp
import numpy as np
from jax.experimental import pallas as pl
from jax.experimental.pallas import tpu as pltpu

B = 4
S = 8192
D = 768
N1 = 64
N2 = 128
TOPK = 10
KPAD = 16
DB = 128
TB2 = 2048
TB3 = 2048


def _const_mats():
    t2 = np.arange(N2, dtype=np.float64)[:, None]
    lg = np.arange(N1, dtype=np.float64)[None, :]
    dft_h = np.concatenate(
        [np.cos(2.0 * np.pi * t2 * lg / N2),
         np.cos(np.pi * t2),
         -np.sin(2.0 * np.pi * t2 * lg[:, 1:] / N2)], axis=1).astype(np.float32)
    t1g = np.arange(N1, dtype=np.float64)[:, None, None]
    lg3 = np.arange(N1, dtype=np.float64)[None, None, :]
    angt = 2.0 * np.pi * t1g * lg3 / S
    c1 = np.cos(angt)
    s1 = np.sin(angt)
    c2 = c1.copy()
    c2[:, :, 0] = 0.0
    pmat = np.concatenate([c1, c2], axis=2).astype(np.float32)
    qmat = np.concatenate([s1, -s1], axis=2).astype(np.float32)
    f1 = np.arange(33, dtype=np.float64)
    t1 = np.arange(N1, dtype=np.float64)
    ang3 = 2.0 * np.pi * np.outer(f1, t1) / N1
    z7 = np.zeros((7, N1))
    cs = np.concatenate([np.cos(ang3), z7, np.sin(ang3), z7],
                        axis=0).astype(np.float32)
    angc = 2.0 * np.pi * np.outer(2.0 * f1 + 1.0, t1) / 128.0
    csc = np.concatenate([np.cos(angc), z7, np.sin(angc), z7],
                         axis=0).astype(np.float32)
    return dft_h, pmat, qmat, cs, csc


_DFTH, _PM, _QM, _CS, _CSC = _const_mats()


def _bsplit(a):
    hi = a.astype(jnp.bfloat16)
    lo = (a - hi.astype(jnp.float32)).astype(jnp.bfloat16)
    return hi, lo


def _dot3(a, b, dims):
    ah, al = _bsplit(a)
    bh, bl = _bsplit(b)

    def dd(u, v):
        return jax.lax.dot_general(u, v, (dims, ((), ())),
                                   preferred_element_type=jnp.float32)

    return dd(ah, bh) + dd(ah, bl) + dd(al, bh)


def _p1_kernel(x_ref, dft_ref, p_ref, q_ref, cs_ref, csc_ref,
               idx_ref, acc_ref):
    j = pl.program_id(1)
    nd = pl.num_programs(1)
    xb = x_ref[0]
    xb2 = xb.reshape(N2, N1 * DB)
    y = _dot3(xb2, dft_ref[...], ((0,), (0,)))
    y3 = y.reshape(N1, DB, N2)
    yp = y3 * p_ref[...] + jnp.roll(y3, N1, axis=2) * q_ref[...]
    mm = _dot3(cs_ref[...], yp, ((1,), (0,)))
    mmr = jnp.roll(mm, N1, axis=2)
    u = mm[:40] + mmr[40:]
    v = mmr[:40] - mm[40:]
    sab = jnp.sum(jnp.sqrt(u * u + v * v), axis=1)
    m3 = _dot3(csc_ref[...], y3[:, :, N1], ((1,), (0,)))
    sc = jnp.sum(jnp.sqrt(m3[:40] ** 2 + m3[40:] ** 2), axis=1)
    s = jnp.concatenate(
        [sab, sc[:, None], jnp.zeros((40, N2 - 1), jnp.float32)], axis=1)

    @pl.when(j == 0)
    def _():
        acc_ref[...] = s

    @pl.when(j > 0)
    def _():
        acc_ref[...] = acc_ref[...] + s

    @pl.when(j == nd - 1)
    def _():
        a = acc_ref[...]
        i0 = jax.lax.broadcasted_iota(jnp.int32, (40, 2 * N2), 0)
        i1 = jax.lax.broadcasted_iota(jnp.int32, (40, 2 * N2), 1)
        fa = N2 * i0 + i1
        fb = N2 * i0 - (i1 - N1)
        fc = N2 * i0 + N1
        fmat = jnp.where(i1 < N1, fa, jnp.where(i1 < 2 * N1, fb, fc))
        valid = ((((i1 < N1) & (fa <= S // 2))
                  | ((i1 >= N1 + 1) & (i1 < 2 * N1) & (i0 >= 1))
                  | ((i1 == 2 * N1) & (i0 <= 31)))
                 & (fmat <= S // 2))
        a = jnp.where(valid, a, -1.0)
        colid = jax.lax.broadcasted_iota(jnp.int32, (1, KPAD), 1)
        row = jnp.zeros((1, KPAD), jnp.int32)
        for k in range(TOPK):
            m = jnp.max(a)
            fk = jnp.min(jnp.where(a == m, fmat, jnp.int32(2 ** 20)))
            row = jnp.where(colid == k, fk, row)
            a = jnp.where(fmat == fk, -1.0, a)
        idx_ref[0] = row


def _basis_block(idx_row, t0, tb, ncols):
    f2x = jnp.concatenate([idx_row, idx_row], axis=1)
    tmat = t0 + jax.lax.broadcasted_iota(jnp.int32, (tb, ncols), 0)
    prod = tmat * f2x
    ang = (prod & (S - 1)).astype(jnp.float32) * (2.0 * np.pi / S)
    colid = jax.lax.broadcasted_iota(jnp.int32, (tb, ncols), 1)
    return jnp.where(colid < KPAD, jnp.cos(ang), jnp.sin(ang)), f2x, colid


def _p2_kernel(x_ref, idx_ref, w_ref, pq_ref, acc_ref):
    j = pl.program_id(1)
    nt = pl.num_programs(1)
    xb = x_ref[0]
    basis, _, _ = _basis_block(idx_ref[0], j * TB2, TB2, 2 * KPAD)
    ps = jax.lax.dot_general(basis.astype(jnp.bfloat16), xb.astype(jnp.bfloat16),
                             (((0,), (0,)), ((), ())),
                             preferred_element_type=jnp.float32)

    @pl.when(j == 0)
    def _():
        acc_ref[...] = ps

    @pl.when(j > 0)
    def _():
        acc_ref[...] = acc_ref[...] + ps

    @pl.when(j == nt - 1)
    def _():
        pq_ref[0] = _dot3(acc_ref[...], w_ref[...], ((1,), (1,)))


def _p3_kernel(pq_ref, idx_ref, bias_ref, y_ref):
    j = pl.program_id(1)
    basis, f2x, colid = _basis_block(idx_ref[0], j * TB3, TB3, 2 * KPAD)
    kid = colid & (KPAD - 1)
    cval = jnp.where((f2x == 0) | (f2x == S // 2), 1.0, 2.0)
    coef = jnp.where(kid < TOPK, cval, 0.0) * (1.0 / S)
    basis = basis * coef
    y = jax.lax.dot_general(basis.astype(jnp.bfloat16),
                            pq_ref[0].astype(jnp.bfloat16),
                            (((1,), (0,)), ((), ())),
                            preferred_element_type=jnp.float32)
    y_ref[0] = y + bias_ref[...]


@jax.jit
def kernel(x, W, b):
    x4 = x.reshape(B, N2, N1, D)
    nd = D // DB
    idx = pl.pallas_call(
        _p1_kernel,
        grid=(B, nd),
        in_specs=[
            pl.BlockSpec((1, N2, N1, DB), lambda bi, j: (bi, 0, 0, j)),
            pl.BlockSpec((N2, N2), lambda bi, j: (0, 0)),
            pl.BlockSpec((N1, 1, N2), lambda bi, j: (0, 0, 0)),
            pl.BlockSpec((N1, 1, N2), lambda bi, j: (0, 0, 0)),
            pl.BlockSpec((80, N1), lambda bi, j: (0, 0)),
            pl.BlockSpec((80, N1), lambda bi, j: (0, 0)),
        ],
        out_specs=pl.BlockSpec((1, 1, KPAD), lambda bi, j: (bi, 0, 0)),
        out_shape=jax.ShapeDtypeStruct((B, 1, KPAD), jnp.int32),
        scratch_shapes=[pltpu.VMEM((40, 2 * N2), jnp.float32)],
        compiler_params=pltpu.CompilerParams(
            dimension_semantics=("arbitrary", "arbitrary")),
    )(x4, _DFTH, _PM, _QM, _CS, _CSC)

    nt = S // TB2
    pq = pl.pallas_call(
        _p2_kernel,
        grid=(B, nt),
        in_specs=[
            pl.BlockSpec((1, TB2, D), lambda bi, j: (bi, j, 0)),
            pl.BlockSpec((1, 1, KPAD), lambda bi, j: (bi, 0, 0)),
            pl.BlockSpec((D, D), lambda bi, j: (0, 0)),
        ],
        out_specs=pl.BlockSpec((1, 2 * KPAD, D), lambda bi, j: (bi, 0, 0)),
        out_shape=jax.ShapeDtypeStruct((B, 2 * KPAD, D), jnp.float32),
        scratch_shapes=[pltpu.VMEM((2 * KPAD, D), jnp.float32)],
        compiler_params=pltpu.CompilerParams(
            dimension_semantics=("arbitrary", "arbitrary")),
    )(x, idx, W)

    nt3 = S // TB3
    y = pl.pallas_call(
        _p3_kernel,
        grid=(B, nt3),
        in_specs=[
            pl.BlockSpec((1, 2 * KPAD, D), lambda bi, j: (bi, 0, 0)),
            pl.BlockSpec((1, 1, KPAD), lambda bi, j: (bi, 0, 0)),
            pl.BlockSpec((1, D), lambda bi, j: (0, 0)),
        ],
        out_specs=pl.BlockSpec((1, TB3, D), lambda bi, j: (bi, j, 0)),
        out_shape=jax.ShapeDtypeStruct((B, S, D), jnp.float32),
        compiler_params=pltpu.CompilerParams(
            dimension_semantics=("arbitrary", "arbitrary")),
    )(pq, idx, b.reshape(1, D))
    return y
